# Initial kernel scaffold; baseline (speedup 1.0000x reference)
import jax
import jax.numpy as jnp
from jax.experimental import pallas as pl


def kernel(x, edge_index, W1, b1, W2, b2, mlp_w, mlp_b, ln_g, ln_b, out_w, out_b):
    raise NotImplementedError("write your pallas kernel here")


# speedup vs baseline: 18.0745x; 18.0745x over previous
import functools

import jax
import jax.numpy as jnp
from jax import lax
from jax.experimental import pallas as pl
from jax.experimental.pallas import tpu as pltpu
from jax.experimental.pallas import tpu_sc as plsc

N = 10000
D = 128
E = 320000
OUT = 4

NC = 2
NS = 16
NW = NC * NS
K = 64
CH = 16
NCH = 10
NB = CH * NCH
EPP = NB * K
EPAD = NW * EPP
NP = 10240
RPT = NP // NS

_MESH = plsc.VectorSubcoreMesh(
    core_axis_name="c", subcore_axis_name="s", num_cores=NC, num_subcores=NS)


def _scatter_body(g_hbm, src_hbm, dst_hbm, zeros_hbm, out_hbm,
                  src_c, dst_c, rows0, rows1, acc_sh, sem0, sem1):
    c = lax.axis_index("c")
    s = lax.axis_index("s")
    wid = s * NC + c
    pltpu.sync_copy(zeros_hbm.at[pl.ds(s * RPT, RPT)],
                    acc_sh.at[pl.ds(s * RPT, RPT)])
    plsc.subcore_barrier()

    def chunk(ci, carry):
        base = wid * NCH + ci
        pltpu.sync_copy(src_hbm.at[base], src_c)
        pltpu.sync_copy(dst_hbm.at[base], dst_c)
        pltpu.async_copy(g_hbm.at[src_c.at[0]], rows0, sem0)

        def pair(p, carry2):
            j0 = 2 * p
            pltpu.async_copy(g_hbm.at[src_c.at[j0 + 1]], rows1, sem1)
            pltpu.make_async_copy(g_hbm.at[src_c.at[j0]], rows0, sem0).wait()
            pltpu.sync_copy(rows0, acc_sh.at[dst_c.at[j0]], add=True)
            pltpu.async_copy(g_hbm.at[src_c.at[j0 + 2]], rows0, sem0)
            pltpu.make_async_copy(g_hbm.at[src_c.at[j0 + 1]], rows1, sem1).wait()
            pltpu.sync_copy(rows1, acc_sh.at[dst_c.at[j0 + 1]], add=True)
            return carry2

        lax.fori_loop(0, CH // 2 - 1, pair, 0)
        j0 = CH - 2
        pltpu.async_copy(g_hbm.at[src_c.at[j0 + 1]], rows1, sem1)
        pltpu.make_async_copy(g_hbm.at[src_c.at[j0]], rows0, sem0).wait()
        pltpu.sync_copy(rows0, acc_sh.at[dst_c.at[j0]], add=True)
        pltpu.make_async_copy(g_hbm.at[src_c.at[j0 + 1]], rows1, sem1).wait()
        pltpu.sync_copy(rows1, acc_sh.at[dst_c.at[j0 + 1]], add=True)
        return carry

    lax.fori_loop(0, NCH, chunk, 0)

    plsc.subcore_barrier()
    pltpu.sync_copy(acc_sh.at[pl.ds(s * RPT, RPT)],
                    out_hbm.at[c, pl.ds(s * RPT, RPT)])


_scatter_kernel = functools.partial(
    pl.kernel,
    out_type=jax.ShapeDtypeStruct((NC, NP, D), jnp.float32),
    mesh=_MESH,
    scratch_types=[
        pltpu.VMEM((CH, K), jnp.int32),
        pltpu.VMEM((CH, K), jnp.int32),
        pltpu.VMEM((K, D), jnp.float32),
        pltpu.VMEM((K, D), jnp.float32),
        pltpu.VMEM_SHARED((NP, D), jnp.float32),
        pltpu.SemaphoreType.DMA,
        pltpu.SemaphoreType.DMA,
    ],
)(_scatter_body)


_BR = 1024


def _stage_a_body(x_ref, w1_ref, d0_ref, d1_ref, g1_ref, dinv_ref):
    deg = d0_ref[:, 0:1] + d1_ref[:, 0:1] + 1.0
    dinv = lax.rsqrt(deg)
    dinv_b = jnp.broadcast_to(dinv, (_BR, D))
    dinv_ref[...] = dinv_b
    g1_ref[...] = jnp.dot(x_ref[...], w1_ref[...],
                          preferred_element_type=jnp.float32) * dinv_b


def _stage_b_body(a0_ref, a1_ref, g1_ref, dinv_ref, w2_ref, b1_ref, g2_ref):
    dinv = dinv_ref[...]
    pre = dinv * (a0_ref[...] + a1_ref[...] + g1_ref[...]) + b1_ref[...]
    h = jax.nn.sigmoid(pre)
    g2_ref[...] = jnp.dot(h, w2_ref[...],
                          preferred_element_type=jnp.float32) * dinv


def _stage_c_body(a0_ref, a1_ref, g2_ref, dinv_ref, b2_ref,
                  mw_ref, mb_ref, lg_ref, lb_ref, ow_ref, ob_ref, out_ref):
    dinv = dinv_ref[...]
    y = jax.nn.relu(dinv * (a0_ref[...] + a1_ref[...] + g2_ref[...])
                    + b2_ref[...])
    z = jnp.dot(y, mw_ref[...], preferred_element_type=jnp.float32) + mb_ref[...]
    mu = jnp.mean(z, axis=-1, keepdims=True)
    var = jnp.mean((z - mu) ** 2, axis=-1, keepdims=True)
    zn = (z - mu) * lax.rsqrt(var + 1e-5) * lg_ref[...] + lb_ref[...]
    o = jnp.dot(zn, ow_ref[...], preferred_element_type=jnp.float32) + ob_ref[...]
    m = jnp.max(o, axis=-1, keepdims=True)
    e = jnp.exp(o - m)
    out_ref[...] = e / jnp.sum(e, axis=-1, keepdims=True)


def _row_spec(width):
    return pl.BlockSpec((_BR, width), lambda i: (i, 0))


def _full_spec(shape):
    return pl.BlockSpec(shape, lambda i: tuple(0 for _ in shape))


def kernel(x, edge_index, W1, b1, W2, b2, mlp_w, mlp_b, ln_g, ln_b, out_w, out_b):
    pad = EPAD - E
    pad_idx = jnp.arange(pad, dtype=jnp.int32)
    pad_src = (pad_idx * 37) % N
    pad_dst = N + pad_idx % (NP - N)
    src_r = jnp.concatenate([edge_index[0], pad_src]).reshape(NW * NCH, CH, K)
    dst_r = jnp.concatenate([edge_index[1], pad_dst]).reshape(NW * NCH, CH, K)
    zeros_nd = jnp.zeros((NP, D), jnp.float32)
    ones_nd = jnp.ones((NP, D), jnp.float32)
    x_pad = jnp.pad(x, ((0, NP - N), (0, 0)))

    accd = _scatter_kernel(ones_nd, src_r, dst_r, zeros_nd)

    grid = NP // _BR
    g1, dinv_b = pl.pallas_call(
        _stage_a_body,
        grid=(grid,),
        in_specs=[_row_spec(D), _full_spec((D, D)), _row_spec(D), _row_spec(D)],
        out_specs=[_row_spec(D), _row_spec(D)],
        out_shape=[jax.ShapeDtypeStruct((NP, D), jnp.float32),
                   jax.ShapeDtypeStruct((NP, D), jnp.float32)],
    )(x_pad, W1, accd[0], accd[1])

    acc1 = _scatter_kernel(g1, src_r, dst_r, zeros_nd)

    g2 = pl.pallas_call(
        _stage_b_body,
        grid=(grid,),
        in_specs=[_row_spec(D), _row_spec(D), _row_spec(D), _row_spec(D),
                  _full_spec((D, D)), _full_spec((1, D))],
        out_specs=_row_spec(D),
        out_shape=jax.ShapeDtypeStruct((NP, D), jnp.float32),
    )(acc1[0], acc1[1], g1, dinv_b, W2, b1.reshape(1, D))

    acc2 = _scatter_kernel(g2, src_r, dst_r, zeros_nd)

    ow_pad = jnp.zeros((D, D), jnp.float32).at[:, :OUT].set(out_w)
    ob_pad = jnp.full((1, D), -1e30, jnp.float32).at[0, :OUT].set(out_b)

    out_pad = pl.pallas_call(
        _stage_c_body,
        grid=(grid,),
        in_specs=[_row_spec(D), _row_spec(D), _row_spec(D), _row_spec(D),
                  _full_spec((1, D)), _full_spec((D, D)), _full_spec((1, D)),
                  _full_spec((1, D)), _full_spec((1, D)), _full_spec((D, D)),
                  _full_spec((1, D))],
        out_specs=_row_spec(D),
        out_shape=jax.ShapeDtypeStruct((NP, D), jnp.float32),
    )(acc2[0], acc2[1], g2, dinv_b, b2.reshape(1, D), mlp_w,
      mlp_b.reshape(1, D), ln_g.reshape(1, D), ln_b.reshape(1, D),
      ow_pad, ob_pad)

    return out_pad[:N, :OUT]


# speedup vs baseline: 22.3198x; 1.2349x over previous
import functools

import jax
import jax.numpy as jnp
from jax import lax
from jax.experimental import pallas as pl
from jax.experimental.pallas import tpu as pltpu
from jax.experimental.pallas import tpu_sc as plsc

N = 10000
D = 128
E = 320000
OUT = 4

NC = 2
NS = 16
NW = NC * NS
K = 128
CH = 8
NCH = 10
NB = CH * NCH
EPP = NB * K
EPAD = NW * EPP
NP = 10240
RPT = NP // NS

_MESH = plsc.VectorSubcoreMesh(
    core_axis_name="c", subcore_axis_name="s", num_cores=NC, num_subcores=NS)


def _deg_body(ones_hbm, dst_hbm, zeros_hbm, out_hbm, dst_c, rows0, acc_sh):
    c = lax.axis_index("c")
    s = lax.axis_index("s")
    wid = s * NC + c
    pltpu.sync_copy(ones_hbm, rows0)
    pltpu.sync_copy(zeros_hbm.at[pl.ds(s * RPT, RPT)],
                    acc_sh.at[pl.ds(s * RPT, RPT)])
    plsc.subcore_barrier()

    def chunk(ci, carry):
        base = wid * NCH + ci
        pltpu.sync_copy(dst_hbm.at[base], dst_c)

        def bj(j, carry2):
            pltpu.sync_copy(rows0, acc_sh.at[dst_c.at[j]], add=True)
            return carry2

        lax.fori_loop(0, CH, bj, 0)
        return carry

    lax.fori_loop(0, NCH, chunk, 0)

    plsc.subcore_barrier()
    pltpu.sync_copy(acc_sh.at[pl.ds(s * RPT, RPT)],
                    out_hbm.at[c, pl.ds(s * RPT, RPT)])


_deg_kernel = functools.partial(
    pl.kernel,
    out_type=jax.ShapeDtypeStruct((NC, NP, D), jnp.float32),
    mesh=_MESH,
    scratch_types=[
        pltpu.VMEM((CH, K), jnp.int32),
        pltpu.VMEM((K, D), jnp.float32),
        pltpu.VMEM_SHARED((NP, D), jnp.float32),
    ],
)(_deg_body)


def _scatter_body(g_hbm, src_hbm, dst_hbm, zeros_hbm, out_hbm,
                  src_c, dst_c, rows0, rows1, acc_sh, sem0, sem1):
    c = lax.axis_index("c")
    s = lax.axis_index("s")
    wid = s * NC + c
    pltpu.sync_copy(zeros_hbm.at[pl.ds(s * RPT, RPT)],
                    acc_sh.at[pl.ds(s * RPT, RPT)])
    plsc.subcore_barrier()

    def chunk(ci, carry):
        base = wid * NCH + ci
        pltpu.sync_copy(src_hbm.at[base], src_c)
        pltpu.sync_copy(dst_hbm.at[base], dst_c)
        pltpu.async_copy(g_hbm.at[src_c.at[0]], rows0, sem0)

        def pair(p, carry2):
            j0 = 2 * p
            pltpu.async_copy(g_hbm.at[src_c.at[j0 + 1]], rows1, sem1)
            pltpu.make_async_copy(g_hbm.at[src_c.at[j0]], rows0, sem0).wait()
            pltpu.sync_copy(rows0, acc_sh.at[dst_c.at[j0]], add=True)
            pltpu.async_copy(g_hbm.at[src_c.at[j0 + 2]], rows0, sem0)
            pltpu.make_async_copy(g_hbm.at[src_c.at[j0 + 1]], rows1, sem1).wait()
            pltpu.sync_copy(rows1, acc_sh.at[dst_c.at[j0 + 1]], add=True)
            return carry2

        lax.fori_loop(0, CH // 2 - 1, pair, 0)
        j0 = CH - 2
        pltpu.async_copy(g_hbm.at[src_c.at[j0 + 1]], rows1, sem1)
        pltpu.make_async_copy(g_hbm.at[src_c.at[j0]], rows0, sem0).wait()
        pltpu.sync_copy(rows0, acc_sh.at[dst_c.at[j0]], add=True)
        pltpu.make_async_copy(g_hbm.at[src_c.at[j0 + 1]], rows1, sem1).wait()
        pltpu.sync_copy(rows1, acc_sh.at[dst_c.at[j0 + 1]], add=True)
        return carry

    lax.fori_loop(0, NCH, chunk, 0)

    plsc.subcore_barrier()
    pltpu.sync_copy(acc_sh.at[pl.ds(s * RPT, RPT)],
                    out_hbm.at[c, pl.ds(s * RPT, RPT)])


_scatter_kernel = functools.partial(
    pl.kernel,
    out_type=jax.ShapeDtypeStruct((NC, NP, D), jnp.float32),
    mesh=_MESH,
    scratch_types=[
        pltpu.VMEM((CH, K), jnp.int32),
        pltpu.VMEM((CH, K), jnp.int32),
        pltpu.VMEM((K, D), jnp.float32),
        pltpu.VMEM((K, D), jnp.float32),
        pltpu.VMEM_SHARED((NP, D), jnp.float32),
        pltpu.SemaphoreType.DMA,
        pltpu.SemaphoreType.DMA,
    ],
)(_scatter_body)


_BR = 1024


def _stage_a_body(x_ref, w1_ref, d0_ref, d1_ref, g1_ref, dinv_ref):
    deg = d0_ref[:, 0:1] + d1_ref[:, 0:1] + 1.0
    dinv = lax.rsqrt(deg)
    dinv_b = jnp.broadcast_to(dinv, (_BR, D))
    dinv_ref[...] = dinv_b
    g1_ref[...] = jnp.dot(x_ref[...], w1_ref[...],
                          preferred_element_type=jnp.float32) * dinv_b


def _stage_b_body(a0_ref, a1_ref, g1_ref, dinv_ref, w2_ref, b1_ref, g2_ref):
    dinv = dinv_ref[...]
    pre = dinv * (a0_ref[...] + a1_ref[...] + g1_ref[...]) + b1_ref[...]
    h = jax.nn.sigmoid(pre)
    g2_ref[...] = jnp.dot(h, w2_ref[...],
                          preferred_element_type=jnp.float32) * dinv


def _stage_c_body(a0_ref, a1_ref, g2_ref, dinv_ref, b2_ref,
                  mw_ref, mb_ref, lg_ref, lb_ref, ow_ref, ob_ref, out_ref):
    dinv = dinv_ref[...]
    y = jax.nn.relu(dinv * (a0_ref[...] + a1_ref[...] + g2_ref[...])
                    + b2_ref[...])
    z = jnp.dot(y, mw_ref[...], preferred_element_type=jnp.float32) + mb_ref[...]
    mu = jnp.mean(z, axis=-1, keepdims=True)
    var = jnp.mean((z - mu) ** 2, axis=-1, keepdims=True)
    zn = (z - mu) * lax.rsqrt(var + 1e-5) * lg_ref[...] + lb_ref[...]
    o = jnp.dot(zn, ow_ref[...], preferred_element_type=jnp.float32) + ob_ref[...]
    m = jnp.max(o, axis=-1, keepdims=True)
    e = jnp.exp(o - m)
    out_ref[...] = e / jnp.sum(e, axis=-1, keepdims=True)


def _row_spec(width):
    return pl.BlockSpec((_BR, width), lambda i: (i, 0))


def _full_spec(shape):
    return pl.BlockSpec(shape, lambda i: tuple(0 for _ in shape))


def kernel(x, edge_index, W1, b1, W2, b2, mlp_w, mlp_b, ln_g, ln_b, out_w, out_b):
    pad = EPAD - E
    pad_idx = jnp.arange(pad, dtype=jnp.int32)
    pad_src = (pad_idx * 37) % N
    pad_dst = N + pad_idx % (NP - N)
    src_r = jnp.concatenate([edge_index[0], pad_src]).reshape(NW * NCH, CH, K)
    dst_r = jnp.concatenate([edge_index[1], pad_dst]).reshape(NW * NCH, CH, K)
    zeros_nd = jnp.zeros((NP, D), jnp.float32)
    ones_kd = jnp.ones((K, D), jnp.float32)
    x_pad = jnp.pad(x, ((0, NP - N), (0, 0)))

    accd = _deg_kernel(ones_kd, dst_r, zeros_nd)

    grid = NP // _BR
    g1, dinv_b = pl.pallas_call(
        _stage_a_body,
        grid=(grid,),
        in_specs=[_row_spec(D), _full_spec((D, D)), _row_spec(D), _row_spec(D)],
        out_specs=[_row_spec(D), _row_spec(D)],
        out_shape=[jax.ShapeDtypeStruct((NP, D), jnp.float32),
                   jax.ShapeDtypeStruct((NP, D), jnp.float32)],
    )(x_pad, W1, accd[0], accd[1])

    acc1 = _scatter_kernel(g1, src_r, dst_r, zeros_nd)

    g2 = pl.pallas_call(
        _stage_b_body,
        grid=(grid,),
        in_specs=[_row_spec(D), _row_spec(D), _row_spec(D), _row_spec(D),
                  _full_spec((D, D)), _full_spec((1, D))],
        out_specs=_row_spec(D),
        out_shape=jax.ShapeDtypeStruct((NP, D), jnp.float32),
    )(acc1[0], acc1[1], g1, dinv_b, W2, b1.reshape(1, D))

    acc2 = _scatter_kernel(g2, src_r, dst_r, zeros_nd)

    ow_pad = jnp.zeros((D, D), jnp.float32).at[:, :OUT].set(out_w)
    ob_pad = jnp.full((1, D), -1e30, jnp.float32).at[0, :OUT].set(out_b)

    out_pad = pl.pallas_call(
        _stage_c_body,
        grid=(grid,),
        in_specs=[_row_spec(D), _row_spec(D), _row_spec(D), _row_spec(D),
                  _full_spec((1, D)), _full_spec((D, D)), _full_spec((1, D)),
                  _full_spec((1, D)), _full_spec((1, D)), _full_spec((D, D)),
                  _full_spec((1, D))],
        out_specs=_row_spec(D),
        out_shape=jax.ShapeDtypeStruct((NP, D), jnp.float32),
    )(acc2[0], acc2[1], g2, dinv_b, b2.reshape(1, D), mlp_w,
      mlp_b.reshape(1, D), ln_g.reshape(1, D), ln_b.reshape(1, D),
      ow_pad, ob_pad)

    return out_pad[:N, :OUT]


# speedup vs baseline: 25.1744x; 1.1279x over previous
import functools

import jax
import jax.numpy as jnp
from jax import lax
from jax.experimental import pallas as pl
from jax.experimental.pallas import tpu as pltpu
from jax.experimental.pallas import tpu_sc as plsc

N = 10000
D = 128
E = 320000
OUT = 4

NC = 2
NS = 16
NW = NC * NS
K = 128
CH = 8
NCH = 10
NB = CH * NCH
EPP = NB * K
EPAD = NW * EPP
NP = 10240
RPT = NP // NS

_MESH = plsc.VectorSubcoreMesh(
    core_axis_name="c", subcore_axis_name="s", num_cores=NC, num_subcores=NS)


def _deg_body(ones_hbm, dst_hbm, zeros_hbm, out_hbm, dst_a, dst_b, rows0,
              acc_sh, semi):
    c = lax.axis_index("c")
    s = lax.axis_index("s")
    wid = s * NC + c
    base0 = wid * NCH
    pltpu.sync_copy(ones_hbm, rows0)
    pltpu.sync_copy(dst_hbm.at[base0], dst_a)
    pltpu.async_copy(dst_hbm.at[base0 + 1], dst_b, semi)
    pltpu.sync_copy(zeros_hbm.at[pl.ds(s * RPT, RPT)],
                    acc_sh.at[pl.ds(s * RPT, RPT)])
    plsc.subcore_barrier()

    def run_chunk(ci, dst_c, dst_n):
        def bj(j, carry2):
            pltpu.sync_copy(rows0, acc_sh.at[dst_c.at[j]], add=True)
            return carry2

        lax.fori_loop(0, CH, bj, 0)

        @pl.when(ci + 1 < NCH)
        def _():
            pltpu.make_async_copy(dst_hbm.at[base0], dst_n, semi).wait()

        @pl.when(ci + 2 < NCH)
        def _():
            pltpu.async_copy(dst_hbm.at[base0 + ci + 2], dst_c, semi)

    def chunk2(c2, carry):
        run_chunk(2 * c2, dst_a, dst_b)
        run_chunk(2 * c2 + 1, dst_b, dst_a)
        return carry

    lax.fori_loop(0, NCH // 2, chunk2, 0)

    plsc.subcore_barrier()
    pltpu.sync_copy(acc_sh.at[pl.ds(s * RPT, RPT)],
                    out_hbm.at[c, pl.ds(s * RPT, RPT)])


_deg_kernel = functools.partial(
    pl.kernel,
    out_type=jax.ShapeDtypeStruct((NC, NP, D), jnp.float32),
    mesh=_MESH,
    scratch_types=[
        pltpu.VMEM((CH, K), jnp.int32),
        pltpu.VMEM((CH, K), jnp.int32),
        pltpu.VMEM((K, D), jnp.float32),
        pltpu.VMEM_SHARED((NP, D), jnp.float32),
        pltpu.SemaphoreType.DMA,
    ],
)(_deg_body)


def _scatter_body(g_hbm, src_hbm, dst_hbm, zeros_hbm, out_hbm,
                  src_a, dst_a, src_b, dst_b, rows0, rows1, acc_sh,
                  sem0, sem1, semi):
    c = lax.axis_index("c")
    s = lax.axis_index("s")
    wid = s * NC + c
    base0 = wid * NCH
    pltpu.sync_copy(src_hbm.at[base0], src_a)
    pltpu.sync_copy(dst_hbm.at[base0], dst_a)
    pltpu.async_copy(src_hbm.at[base0 + 1], src_b, semi)
    pltpu.async_copy(dst_hbm.at[base0 + 1], dst_b, semi)
    pltpu.sync_copy(zeros_hbm.at[pl.ds(s * RPT, RPT)],
                    acc_sh.at[pl.ds(s * RPT, RPT)])
    plsc.subcore_barrier()
    pltpu.async_copy(g_hbm.at[src_a.at[0]], rows0, sem0)

    def run_chunk(ci, src_c, dst_c, src_n, dst_n):
        def pair(p, carry2):
            j0 = 2 * p
            pltpu.async_copy(g_hbm.at[src_c.at[j0 + 1]], rows1, sem1)
            pltpu.make_async_copy(g_hbm.at[src_c.at[j0]], rows0, sem0).wait()
            pltpu.sync_copy(rows0, acc_sh.at[dst_c.at[j0]], add=True)
            pltpu.async_copy(g_hbm.at[src_c.at[j0 + 2]], rows0, sem0)
            pltpu.make_async_copy(g_hbm.at[src_c.at[j0 + 1]], rows1, sem1).wait()
            pltpu.sync_copy(rows1, acc_sh.at[dst_c.at[j0 + 1]], add=True)
            return carry2

        lax.fori_loop(0, CH // 2 - 1, pair, 0)
        j0 = CH - 2
        pltpu.async_copy(g_hbm.at[src_c.at[j0 + 1]], rows1, sem1)
        pltpu.make_async_copy(g_hbm.at[src_c.at[j0]], rows0, sem0).wait()
        pltpu.sync_copy(rows0, acc_sh.at[dst_c.at[j0]], add=True)

        @pl.when(ci + 1 < NCH)
        def _():
            pltpu.make_async_copy(src_hbm.at[base0], src_n, semi).wait()
            pltpu.make_async_copy(dst_hbm.at[base0], dst_n, semi).wait()
            pltpu.async_copy(g_hbm.at[src_n.at[0]], rows0, sem0)

        pltpu.make_async_copy(g_hbm.at[src_c.at[j0 + 1]], rows1, sem1).wait()
        pltpu.sync_copy(rows1, acc_sh.at[dst_c.at[j0 + 1]], add=True)

        @pl.when(ci + 2 < NCH)
        def _():
            pltpu.async_copy(src_hbm.at[base0 + ci + 2], src_c, semi)
            pltpu.async_copy(dst_hbm.at[base0 + ci + 2], dst_c, semi)

    def chunk2(c2, carry):
        run_chunk(2 * c2, src_a, dst_a, src_b, dst_b)
        run_chunk(2 * c2 + 1, src_b, dst_b, src_a, dst_a)
        return carry

    lax.fori_loop(0, NCH // 2, chunk2, 0)

    plsc.subcore_barrier()
    pltpu.sync_copy(acc_sh.at[pl.ds(s * RPT, RPT)],
                    out_hbm.at[c, pl.ds(s * RPT, RPT)])


_scatter_kernel = functools.partial(
    pl.kernel,
    out_type=jax.ShapeDtypeStruct((NC, NP, D), jnp.float32),
    mesh=_MESH,
    scratch_types=[
        pltpu.VMEM((CH, K), jnp.int32),
        pltpu.VMEM((CH, K), jnp.int32),
        pltpu.VMEM((CH, K), jnp.int32),
        pltpu.VMEM((CH, K), jnp.int32),
        pltpu.VMEM((K, D), jnp.float32),
        pltpu.VMEM((K, D), jnp.float32),
        pltpu.VMEM_SHARED((NP, D), jnp.float32),
        pltpu.SemaphoreType.DMA,
        pltpu.SemaphoreType.DMA,
        pltpu.SemaphoreType.DMA,
    ],
)(_scatter_body)


_BR = 1024


def _stage_a_body(x_ref, w1_ref, d0_ref, d1_ref, g1_ref, dinv_ref):
    deg = d0_ref[:, 0:1] + d1_ref[:, 0:1] + 1.0
    dinv = lax.rsqrt(deg)
    dinv_b = jnp.broadcast_to(dinv, (_BR, D))
    dinv_ref[...] = dinv_b
    g1_ref[...] = jnp.dot(x_ref[...], w1_ref[...],
                          preferred_element_type=jnp.float32) * dinv_b


def _stage_b_body(a0_ref, a1_ref, g1_ref, dinv_ref, w2_ref, b1_ref, g2_ref):
    dinv = dinv_ref[...]
    pre = dinv * (a0_ref[...] + a1_ref[...] + g1_ref[...]) + b1_ref[...]
    h = jax.nn.sigmoid(pre)
    g2_ref[...] = jnp.dot(h, w2_ref[...],
                          preferred_element_type=jnp.float32) * dinv


def _stage_c_body(a0_ref, a1_ref, g2_ref, dinv_ref, b2_ref,
                  mw_ref, mb_ref, lg_ref, lb_ref, ow_ref, ob_ref, out_ref):
    dinv = dinv_ref[...]
    y = jax.nn.relu(dinv * (a0_ref[...] + a1_ref[...] + g2_ref[...])
                    + b2_ref[...])
    z = jnp.dot(y, mw_ref[...], preferred_element_type=jnp.float32) + mb_ref[...]
    mu = jnp.mean(z, axis=-1, keepdims=True)
    var = jnp.mean((z - mu) ** 2, axis=-1, keepdims=True)
    zn = (z - mu) * lax.rsqrt(var + 1e-5) * lg_ref[...] + lb_ref[...]
    o = jnp.dot(zn, ow_ref[...], preferred_element_type=jnp.float32) + ob_ref[...]
    m = jnp.max(o, axis=-1, keepdims=True)
    e = jnp.exp(o - m)
    out_ref[...] = e / jnp.sum(e, axis=-1, keepdims=True)


def _row_spec(width):
    return pl.BlockSpec((_BR, width), lambda i: (i, 0))


def _full_spec(shape):
    return pl.BlockSpec(shape, lambda i: tuple(0 for _ in shape))


def kernel(x, edge_index, W1, b1, W2, b2, mlp_w, mlp_b, ln_g, ln_b, out_w, out_b):
    pad = EPAD - E
    pad_idx = jnp.arange(pad, dtype=jnp.int32)
    pad_src = (pad_idx * 37) % N
    pad_dst = N + pad_idx % (NP - N)
    src_r = jnp.concatenate([edge_index[0], pad_src]).reshape(NW * NCH, CH, K)
    dst_r = jnp.concatenate([edge_index[1], pad_dst]).reshape(NW * NCH, CH, K)
    zeros_nd = jnp.zeros((NP, D), jnp.float32)
    ones_kd = jnp.ones((K, D), jnp.float32)
    x_pad = jnp.pad(x, ((0, NP - N), (0, 0)))

    accd = _deg_kernel(ones_kd, dst_r, zeros_nd)

    grid = NP // _BR
    g1, dinv_b = pl.pallas_call(
        _stage_a_body,
        grid=(grid,),
        in_specs=[_row_spec(D), _full_spec((D, D)), _row_spec(D), _row_spec(D)],
        out_specs=[_row_spec(D), _row_spec(D)],
        out_shape=[jax.ShapeDtypeStruct((NP, D), jnp.float32),
                   jax.ShapeDtypeStruct((NP, D), jnp.float32)],
    )(x_pad, W1, accd[0], accd[1])

    acc1 = _scatter_kernel(g1, src_r, dst_r, zeros_nd)

    g2 = pl.pallas_call(
        _stage_b_body,
        grid=(grid,),
        in_specs=[_row_spec(D), _row_spec(D), _row_spec(D), _row_spec(D),
                  _full_spec((D, D)), _full_spec((1, D))],
        out_specs=_row_spec(D),
        out_shape=jax.ShapeDtypeStruct((NP, D), jnp.float32),
    )(acc1[0], acc1[1], g1, dinv_b, W2, b1.reshape(1, D))

    acc2 = _scatter_kernel(g2, src_r, dst_r, zeros_nd)

    ow_pad = jnp.zeros((D, D), jnp.float32).at[:, :OUT].set(out_w)
    ob_pad = jnp.full((1, D), -1e30, jnp.float32).at[0, :OUT].set(out_b)

    out_pad = pl.pallas_call(
        _stage_c_body,
        grid=(grid,),
        in_specs=[_row_spec(D), _row_spec(D), _row_spec(D), _row_spec(D),
                  _full_spec((1, D)), _full_spec((D, D)), _full_spec((1, D)),
                  _full_spec((1, D)), _full_spec((1, D)), _full_spec((D, D)),
                  _full_spec((1, D))],
        out_specs=_row_spec(D),
        out_shape=jax.ShapeDtypeStruct((NP, D), jnp.float32),
    )(acc2[0], acc2[1], g2, dinv_b, b2.reshape(1, D), mlp_w,
      mlp_b.reshape(1, D), ln_g.reshape(1, D), ln_b.reshape(1, D),
      ow_pad, ob_pad)

    return out_pad[:N, :OUT]
